# baseline (speedup 1.0000x reference)
import numpy as np
import jax
import jax.numpy as jnp
from jax.experimental import pallas as pl
from jax.experimental.pallas import tpu as pltpu

N = 10000
H = 256
B = 200
TD_MAX = 100.0
NPAD = 10240
BR = 8


def _scan_kernel(u_ref, v_ref, et_ref, td_ref, z0_ref, wb_ref,
                 Wbig_ref, bias_ref,
                 srow_ref, a2_ref, z_ref, s_ref):
    i = pl.program_id(0)

    @pl.when(i == 0)
    def _init():
        z_ref[...] = z0_ref[...]
        s_ref[:, 0:N] = jax.lax.dot_general(
            wb_ref[...], z0_ref[...],
            dimension_numbers=(((1,), (1,)), ((), ())),
            preferred_element_type=jnp.float32)
        s_ref[:, N:NPAD] = jnp.zeros((2, NPAD - N), jnp.float32)

    ui = u_ref[i]
    vi = v_ref[i]
    et = et_ref[i]

    zu = z_ref[pl.ds(ui, 1), :]
    zv = z_ref[pl.ds(vi, 1), :]

    r = i % 8
    srow_ref[pl.ds(r, 1), :] = s_ref[pl.ds(et, 1), 0:N]

    tdn = td_ref[pl.ds(i, 1), :]
    mm = jnp.concatenate(
        [jnp.concatenate([zv, zu, tdn[:, 0:4]], axis=1),
         jnp.concatenate([zu, zv, tdn[:, 4:8]], axis=1)], axis=0)
    out = jnp.dot(mm, Wbig_ref[...],
                  preferred_element_type=jnp.float32)
    h2 = jax.nn.sigmoid(out[:, 0:H] + bias_ref[...])
    a_col = jnp.where(et == 0, out[:, H:H + 1], out[:, H + 1:H + 2])
    a2_ref[pl.ds(r, 1), :] = jnp.concatenate(
        [a_col[0:1, :], a_col[1:2, :]], axis=1)

    hu = h2[0:1, :]
    hv = h2[1:2, :]
    z_ref[pl.ds(ui, 1), :] = hu
    z_ref[pl.ds(vi, 1), :] = hv
    val_u = jnp.sum(wb_ref[...] * hu, axis=1, keepdims=True)
    val_v = jnp.sum(wb_ref[...] * hv, axis=1, keepdims=True)
    lane = jax.lax.broadcasted_iota(jnp.int32, (1, 128), 1)
    base_u = pl.multiple_of((ui // 128) * 128, 128)
    tile_u = s_ref[:, pl.ds(base_u, 128)]
    s_ref[:, pl.ds(base_u, 128)] = jnp.where(lane == ui % 128, val_u, tile_u)
    base_v = pl.multiple_of((vi // 128) * 128, 128)
    tile_v = s_ref[:, pl.ds(base_v, 128)]
    s_ref[:, pl.ds(base_v, 128)] = jnp.where(lane == vi % 128, val_v, tile_v)


def _lam_kernel(wt_ref, al_ref, ps_ref, ob_ref, uv_ref, etc_ref, tf_ref,
                tb_ref, srow_ref, a2_ref, lam_ref):
    u_col = uv_ref[:, 0:1]
    v_col = uv_ref[:, 1:2]
    et_col = etc_ref[...]
    t_col = tf_ref[...]

    et0 = et_col == 0
    wt_col = jnp.where(et0, wt_ref[0], wt_ref[1])
    al_col = jnp.where(et0, al_ref[0], al_ref[1])
    ps_col = jnp.where(et0, ps_ref[0], ps_ref[1])
    ob_col = jnp.where(et0, ob_ref[0], ob_ref[1])
    c1 = wt_col * (1.0 / TD_MAX)
    c2 = jnp.log(al_col) - c1 * t_col
    inv_col = 1.0 / (ps_col + 1e-7)

    tb = tb_ref[:, 0, :]
    iota = jax.lax.broadcasted_iota(jnp.int32, (1, N), 1)
    tbu = jnp.sum(jnp.where(iota == u_col, tb, 0.0), axis=1, keepdims=True)
    tbv = jnp.sum(jnp.where(iota == v_col, tb, 0.0), axis=1, keepdims=True)
    dec_u = jnp.exp(c1 * jnp.maximum(tbu, tb) + c2)
    dec_v = jnp.exp(c1 * jnp.maximum(tbv, tb) + c2)
    base = srow_ref[...] + ob_col
    g_u = a2_ref[:, 0:1] + base + dec_u
    g_v = a2_ref[:, 1:2] + base + dec_v
    lam_ref[:, 0:N] = ps_col * jnp.log1p(
        jnp.exp(jnp.clip(g_u * inv_col, -75.0, 75.0)))
    lam_ref[:, N:2 * N] = ps_col * jnp.log1p(
        jnp.exp(jnp.clip(g_v * inv_col, -75.0, 75.0)))


@jax.jit
def kernel(u, v, time_diff, event_types, t_bar, t, z0, omega_w, omega_b,
           w_t, alpha, psi, W_struct_w, W_struct_b, W_rec_w, W_rec_b,
           W_t_w, W_t_b):
    u = u.astype(jnp.int32)
    v = v.astype(jnp.int32)
    et = event_types.astype(jnp.int32)
    td8 = time_diff.reshape(B, 8)
    wa = omega_w[:, 0, :H]
    wb = omega_w[:, 0, H:]
    ob = omega_b[:, 0]
    bias = (W_struct_b + W_rec_b + W_t_b).reshape(1, H)
    sd4 = jnp.array([50.0, 7.0, 15.0, 15.0], dtype=jnp.float32)
    WtT_n = W_t_w.T / sd4[:, None]
    Wbig = jnp.zeros((2 * H + 4, H + 2), jnp.float32)
    Wbig = Wbig.at[0:H, 0:H].set(W_struct_w.T)
    Wbig = Wbig.at[H:2 * H, 0:H].set(W_rec_w.T)
    Wbig = Wbig.at[2 * H:2 * H + 4, 0:H].set(WtT_n)
    Wbig = Wbig.at[H:2 * H, H].set(wa[0])
    Wbig = Wbig.at[H:2 * H, H + 1].set(wa[1])

    smem = pl.BlockSpec(memory_space=pltpu.SMEM)
    full = pl.BlockSpec(memory_space=pltpu.VMEM)

    srow2, a2, z_final = pl.pallas_call(
        _scan_kernel,
        grid=(B,),
        in_specs=[
            smem,
            smem,
            smem,
            full,
            full,
            full,
            full,
            full,
        ],
        out_specs=[
            pl.BlockSpec((8, N), lambda i: (i // 8, 0)),
            pl.BlockSpec((8, 2), lambda i: (i // 8, 0)),
            pl.BlockSpec((N, H), lambda i: (0, 0)),
        ],
        out_shape=[
            jax.ShapeDtypeStruct((B, N), jnp.float32),
            jax.ShapeDtypeStruct((B, 2), jnp.float32),
            jax.ShapeDtypeStruct((N, H), jnp.float32),
        ],
        scratch_shapes=[pltpu.VMEM((2, NPAD), jnp.float32)],
        compiler_params=pltpu.CompilerParams(
            dimension_semantics=("arbitrary",)),
    )(u, v, et, td8, z0, wb, Wbig, bias)

    uv = jnp.stack([u, v], axis=1)
    lam = pl.pallas_call(
        _lam_kernel,
        grid=(B // BR,),
        in_specs=[
            smem,
            smem,
            smem,
            smem,
            pl.BlockSpec((BR, 2), lambda b: (b, 0)),
            pl.BlockSpec((BR, 1), lambda b: (b, 0)),
            pl.BlockSpec((BR, 1), lambda b: (b, 0)),
            pl.BlockSpec((BR, 1, N), lambda b: (b, 0, 0)),
            pl.BlockSpec((BR, N), lambda b: (b, 0)),
            pl.BlockSpec((BR, 2), lambda b: (b, 0)),
        ],
        out_specs=pl.BlockSpec((BR, 2 * N), lambda b: (b, 0)),
        out_shape=jax.ShapeDtypeStruct((B, 2 * N), jnp.float32),
        compiler_params=pltpu.CompilerParams(
            dimension_semantics=("parallel",)),
    )(w_t, alpha, psi, ob, uv, et.reshape(B, 1), t.reshape(B, 1),
      t_bar.reshape(B, 1, N), srow2, a2)

    return lam, z_final


# speedup vs baseline: 1.1354x; 1.1354x over previous
import numpy as np
import jax
import jax.numpy as jnp
from jax.experimental import pallas as pl
from jax.experimental.pallas import tpu as pltpu

N = 10000
H = 256
B = 200
TD_MAX = 100.0
NPAD = 10240
BR = 8


def _scan_kernel(u_ref, v_ref, et_ref, td_ref, z0_ref, wa_ref, wb_ref,
                 WsT_ref, WrT_ref, WtT_ref, bias_ref,
                 srow_ref, a2_ref, z_ref, s_ref):
    i = pl.program_id(0)

    @pl.when(i == 0)
    def _init():
        z_ref[...] = z0_ref[...]
        s_ref[:, 0:N] = jax.lax.dot_general(
            wb_ref[...], z0_ref[...],
            dimension_numbers=(((1,), (1,)), ((), ())),
            preferred_element_type=jnp.float32)
        s_ref[:, N:NPAD] = jnp.zeros((2, NPAD - N), jnp.float32)

    ui = u_ref[i]
    vi = v_ref[i]
    et = et_ref[i]

    zu = z_ref[pl.ds(ui, 1), :]
    zv = z_ref[pl.ds(vi, 1), :]

    r = i % 8
    srow_ref[pl.ds(r, 1), :] = s_ref[pl.ds(et, 1), 0:N]
    wa_et = wa_ref[pl.ds(et, 1), :]
    a_u = jnp.sum(zu * wa_et, axis=1, keepdims=True)
    a_v = jnp.sum(zv * wa_et, axis=1, keepdims=True)
    a2_ref[pl.ds(r, 1), :] = jnp.concatenate([a_u, a_v], axis=1)

    tdn = td_ref[pl.ds(i, 1), :]
    td01 = jnp.concatenate([tdn[:, 0:4], tdn[:, 4:8]],
                           axis=0).astype(jnp.bfloat16)
    m_struct = jnp.concatenate([zv, zu], axis=0).astype(jnp.bfloat16)
    m_rec = jnp.concatenate([zu, zv], axis=0).astype(jnp.bfloat16)
    h2 = jax.nn.sigmoid(
        jnp.dot(m_struct, WsT_ref[...], preferred_element_type=jnp.float32)
        + jnp.dot(m_rec, WrT_ref[...], preferred_element_type=jnp.float32)
        + jnp.dot(td01, WtT_ref[...], preferred_element_type=jnp.float32)
        + bias_ref[...])

    hu = h2[0:1, :]
    hv = h2[1:2, :]
    z_ref[pl.ds(ui, 1), :] = hu
    z_ref[pl.ds(vi, 1), :] = hv
    val_u = jnp.sum(wb_ref[...] * hu, axis=1, keepdims=True)
    val_v = jnp.sum(wb_ref[...] * hv, axis=1, keepdims=True)
    lane = jax.lax.broadcasted_iota(jnp.int32, (1, 128), 1)
    base_u = pl.multiple_of((ui // 128) * 128, 128)
    tile_u = s_ref[:, pl.ds(base_u, 128)]
    s_ref[:, pl.ds(base_u, 128)] = jnp.where(lane == ui % 128, val_u, tile_u)
    base_v = pl.multiple_of((vi // 128) * 128, 128)
    tile_v = s_ref[:, pl.ds(base_v, 128)]
    s_ref[:, pl.ds(base_v, 128)] = jnp.where(lane == vi % 128, val_v, tile_v)


def _lam_kernel(wt_ref, al_ref, ps_ref, ob_ref, uv_ref, etc_ref, tf_ref,
                tb_ref, srow_ref, a2_ref, lam_ref):
    u_col = uv_ref[:, 0:1]
    v_col = uv_ref[:, 1:2]
    et_col = etc_ref[...]
    t_col = tf_ref[...]

    et0 = et_col == 0
    wt_col = jnp.where(et0, wt_ref[0], wt_ref[1])
    al_col = jnp.where(et0, al_ref[0], al_ref[1])
    ps_col = jnp.where(et0, ps_ref[0], ps_ref[1])
    ob_col = jnp.where(et0, ob_ref[0], ob_ref[1])
    c1 = wt_col * (1.0 / TD_MAX)
    c2 = jnp.log(al_col) - c1 * t_col
    inv_col = 1.0 / (ps_col + 1e-7)

    tb = tb_ref[:, 0, :]
    iota = jax.lax.broadcasted_iota(jnp.int32, (1, N), 1)
    tbu = jnp.sum(jnp.where(iota == u_col, tb, 0.0), axis=1, keepdims=True)
    tbv = jnp.sum(jnp.where(iota == v_col, tb, 0.0), axis=1, keepdims=True)
    dec_u = jnp.exp(c1 * jnp.maximum(tbu, tb) + c2)
    dec_v = jnp.exp(c1 * jnp.maximum(tbv, tb) + c2)
    base = srow_ref[...] + ob_col
    g_u = a2_ref[:, 0:1] + base + dec_u
    g_v = a2_ref[:, 1:2] + base + dec_v
    lam_ref[:, 0:N] = ps_col * jnp.log1p(
        jnp.exp(jnp.clip(g_u * inv_col, -75.0, 75.0)))
    lam_ref[:, N:2 * N] = ps_col * jnp.log1p(
        jnp.exp(jnp.clip(g_v * inv_col, -75.0, 75.0)))


@jax.jit
def kernel(u, v, time_diff, event_types, t_bar, t, z0, omega_w, omega_b,
           w_t, alpha, psi, W_struct_w, W_struct_b, W_rec_w, W_rec_b,
           W_t_w, W_t_b):
    u = u.astype(jnp.int32)
    v = v.astype(jnp.int32)
    et = event_types.astype(jnp.int32)
    td8 = time_diff.reshape(B, 8)
    wa = omega_w[:, 0, :H]
    wb = omega_w[:, 0, H:]
    ob = omega_b[:, 0]
    bias = (W_struct_b + W_rec_b + W_t_b).reshape(1, H)
    sd4 = jnp.array([50.0, 7.0, 15.0, 15.0], dtype=jnp.float32)
    WtT_n = W_t_w.T / sd4[:, None]
    WsT16 = W_struct_w.T.astype(jnp.bfloat16)
    WrT16 = W_rec_w.T.astype(jnp.bfloat16)
    WtT16 = WtT_n.astype(jnp.bfloat16)

    smem = pl.BlockSpec(memory_space=pltpu.SMEM)
    full = pl.BlockSpec(memory_space=pltpu.VMEM)

    srow2, a2, z_final = pl.pallas_call(
        _scan_kernel,
        grid=(B,),
        in_specs=[
            smem,
            smem,
            smem,
            full,
            full,
            full,
            full,
            full,
            full,
            full,
            full,
        ],
        out_specs=[
            pl.BlockSpec((8, N), lambda i: (i // 8, 0)),
            pl.BlockSpec((8, 2), lambda i: (i // 8, 0)),
            pl.BlockSpec((N, H), lambda i: (0, 0)),
        ],
        out_shape=[
            jax.ShapeDtypeStruct((B, N), jnp.float32),
            jax.ShapeDtypeStruct((B, 2), jnp.float32),
            jax.ShapeDtypeStruct((N, H), jnp.float32),
        ],
        scratch_shapes=[pltpu.VMEM((2, NPAD), jnp.float32)],
        compiler_params=pltpu.CompilerParams(
            dimension_semantics=("arbitrary",)),
    )(u, v, et, td8, z0, wa, wb, WsT16, WrT16, WtT16, bias)

    uv = jnp.stack([u, v], axis=1)
    lam = pl.pallas_call(
        _lam_kernel,
        grid=(B // BR,),
        in_specs=[
            smem,
            smem,
            smem,
            smem,
            pl.BlockSpec((BR, 2), lambda b: (b, 0)),
            pl.BlockSpec((BR, 1), lambda b: (b, 0)),
            pl.BlockSpec((BR, 1), lambda b: (b, 0)),
            pl.BlockSpec((BR, 1, N), lambda b: (b, 0, 0)),
            pl.BlockSpec((BR, N), lambda b: (b, 0)),
            pl.BlockSpec((BR, 2), lambda b: (b, 0)),
        ],
        out_specs=pl.BlockSpec((BR, 2 * N), lambda b: (b, 0)),
        out_shape=jax.ShapeDtypeStruct((B, 2 * N), jnp.float32),
        compiler_params=pltpu.CompilerParams(
            dimension_semantics=("parallel",)),
    )(w_t, alpha, psi, ob, uv, et.reshape(B, 1), t.reshape(B, 1),
      t_bar.reshape(B, 1, N), srow2, a2)

    return lam, z_final


# speedup vs baseline: 1.3381x; 1.1785x over previous
import numpy as np
import jax
import jax.numpy as jnp
from jax.experimental import pallas as pl
from jax.experimental.pallas import tpu as pltpu

N = 10000
H = 256
B = 200
TD_MAX = 100.0
NPAD = 10240
BR = 8


def _scan_kernel(u_ref, v_ref, et_ref, td_ref, z0_ref, wa_ref, wb_ref,
                 WsT_ref, WrT_ref, WtT_ref, bias_ref,
                 srow_ref, a2_ref, z_ref, s_ref):
    j = pl.program_id(0)

    @pl.when(j == 0)
    def _init():
        z_ref[...] = z0_ref[...]
        s_ref[:, 0:N] = jax.lax.dot_general(
            wb_ref[...], z0_ref[...],
            dimension_numbers=(((1,), (1,)), ((), ())),
            preferred_element_type=jnp.float32)
        s_ref[:, N:NPAD] = jnp.zeros((2, NPAD - N), jnp.float32)

    def _event(i, r):
        ui = u_ref[i]
        vi = v_ref[i]
        et = et_ref[i]

        zu = z_ref[pl.ds(ui, 1), :]
        zv = z_ref[pl.ds(vi, 1), :]

        srow_ref[pl.ds(r, 1), :] = s_ref[pl.ds(et, 1), 0:N]
        wa_et = wa_ref[pl.ds(et, 1), :]
        a_u = jnp.sum(zu * wa_et, axis=1, keepdims=True)
        a_v = jnp.sum(zv * wa_et, axis=1, keepdims=True)
        a2_ref[pl.ds(r, 1), :] = jnp.concatenate([a_u, a_v], axis=1)

        tdn = td_ref[pl.ds(i, 1), :]
        td01 = jnp.concatenate([tdn[:, 0:4], tdn[:, 4:8]],
                               axis=0).astype(jnp.bfloat16)
        m_struct = jnp.concatenate([zv, zu], axis=0).astype(jnp.bfloat16)
        m_rec = jnp.concatenate([zu, zv], axis=0).astype(jnp.bfloat16)
        h2 = jax.nn.sigmoid(
            jnp.dot(m_struct, WsT_ref[...],
                    preferred_element_type=jnp.float32)
            + jnp.dot(m_rec, WrT_ref[...],
                      preferred_element_type=jnp.float32)
            + jnp.dot(td01, WtT_ref[...],
                      preferred_element_type=jnp.float32)
            + bias_ref[...])

        hu = h2[0:1, :]
        hv = h2[1:2, :]
        z_ref[pl.ds(ui, 1), :] = hu
        z_ref[pl.ds(vi, 1), :] = hv
        val_u = jnp.sum(wb_ref[...] * hu, axis=1, keepdims=True)
        val_v = jnp.sum(wb_ref[...] * hv, axis=1, keepdims=True)
        lane = jax.lax.broadcasted_iota(jnp.int32, (1, 128), 1)
        base_u = pl.multiple_of((ui // 128) * 128, 128)
        tile_u = s_ref[:, pl.ds(base_u, 128)]
        s_ref[:, pl.ds(base_u, 128)] = jnp.where(
            lane == ui % 128, val_u, tile_u)
        base_v = pl.multiple_of((vi // 128) * 128, 128)
        tile_v = s_ref[:, pl.ds(base_v, 128)]
        s_ref[:, pl.ds(base_v, 128)] = jnp.where(
            lane == vi % 128, val_v, tile_v)

    i0 = 2 * j
    r0 = i0 % 8
    _event(i0, r0)
    _event(i0 + 1, r0 + 1)


def _lam_kernel(wt_ref, al_ref, ps_ref, ob_ref, uv_ref, etc_ref, tf_ref,
                tb_ref, srow_ref, a2_ref, lam_ref):
    u_col = uv_ref[:, 0:1]
    v_col = uv_ref[:, 1:2]
    et_col = etc_ref[...]
    t_col = tf_ref[...]

    et0 = et_col == 0
    wt_col = jnp.where(et0, wt_ref[0], wt_ref[1])
    al_col = jnp.where(et0, al_ref[0], al_ref[1])
    ps_col = jnp.where(et0, ps_ref[0], ps_ref[1])
    ob_col = jnp.where(et0, ob_ref[0], ob_ref[1])
    c1 = wt_col * (1.0 / TD_MAX)
    c2 = jnp.log(al_col) - c1 * t_col
    inv_col = 1.0 / (ps_col + 1e-7)

    tb = tb_ref[:, 0, :]
    iota = jax.lax.broadcasted_iota(jnp.int32, (1, N), 1)
    tbu = jnp.sum(jnp.where(iota == u_col, tb, 0.0), axis=1, keepdims=True)
    tbv = jnp.sum(jnp.where(iota == v_col, tb, 0.0), axis=1, keepdims=True)
    dec_u = jnp.exp(c1 * jnp.maximum(tbu, tb) + c2)
    dec_v = jnp.exp(c1 * jnp.maximum(tbv, tb) + c2)
    base = srow_ref[...] + ob_col
    g_u = a2_ref[:, 0:1] + base + dec_u
    g_v = a2_ref[:, 1:2] + base + dec_v
    lam_ref[:, 0:N] = ps_col * jnp.log1p(
        jnp.exp(jnp.clip(g_u * inv_col, -75.0, 75.0)))
    lam_ref[:, N:2 * N] = ps_col * jnp.log1p(
        jnp.exp(jnp.clip(g_v * inv_col, -75.0, 75.0)))


@jax.jit
def kernel(u, v, time_diff, event_types, t_bar, t, z0, omega_w, omega_b,
           w_t, alpha, psi, W_struct_w, W_struct_b, W_rec_w, W_rec_b,
           W_t_w, W_t_b):
    u = u.astype(jnp.int32)
    v = v.astype(jnp.int32)
    et = event_types.astype(jnp.int32)
    td8 = time_diff.reshape(B, 8)
    wa = omega_w[:, 0, :H]
    wb = omega_w[:, 0, H:]
    ob = omega_b[:, 0]
    bias = (W_struct_b + W_rec_b + W_t_b).reshape(1, H)
    sd4 = jnp.array([50.0, 7.0, 15.0, 15.0], dtype=jnp.float32)
    WtT_n = W_t_w.T / sd4[:, None]
    WsT16 = W_struct_w.T.astype(jnp.bfloat16)
    WrT16 = W_rec_w.T.astype(jnp.bfloat16)
    WtT16 = WtT_n.astype(jnp.bfloat16)

    smem = pl.BlockSpec(memory_space=pltpu.SMEM)
    full = pl.BlockSpec(memory_space=pltpu.VMEM)

    srow2, a2, z_final = pl.pallas_call(
        _scan_kernel,
        grid=(B // 2,),
        in_specs=[
            smem,
            smem,
            smem,
            full,
            full,
            full,
            full,
            full,
            full,
            full,
            full,
        ],
        out_specs=[
            pl.BlockSpec((8, N), lambda j: (j // 4, 0)),
            pl.BlockSpec((8, 2), lambda j: (j // 4, 0)),
            pl.BlockSpec((N, H), lambda j: (0, 0)),
        ],
        out_shape=[
            jax.ShapeDtypeStruct((B, N), jnp.float32),
            jax.ShapeDtypeStruct((B, 2), jnp.float32),
            jax.ShapeDtypeStruct((N, H), jnp.float32),
        ],
        scratch_shapes=[pltpu.VMEM((2, NPAD), jnp.float32)],
        compiler_params=pltpu.CompilerParams(
            dimension_semantics=("arbitrary",)),
    )(u, v, et, td8, z0, wa, wb, WsT16, WrT16, WtT16, bias)

    uv = jnp.stack([u, v], axis=1)
    lam = pl.pallas_call(
        _lam_kernel,
        grid=(B // BR,),
        in_specs=[
            smem,
            smem,
            smem,
            smem,
            pl.BlockSpec((BR, 2), lambda b: (b, 0)),
            pl.BlockSpec((BR, 1), lambda b: (b, 0)),
            pl.BlockSpec((BR, 1), lambda b: (b, 0)),
            pl.BlockSpec((BR, 1, N), lambda b: (b, 0, 0)),
            pl.BlockSpec((BR, N), lambda b: (b, 0)),
            pl.BlockSpec((BR, 2), lambda b: (b, 0)),
        ],
        out_specs=pl.BlockSpec((BR, 2 * N), lambda b: (b, 0)),
        out_shape=jax.ShapeDtypeStruct((B, 2 * N), jnp.float32),
        compiler_params=pltpu.CompilerParams(
            dimension_semantics=("parallel",)),
    )(w_t, alpha, psi, ob, uv, et.reshape(B, 1), t.reshape(B, 1),
      t_bar.reshape(B, 1, N), srow2, a2)

    return lam, z_final


# speedup vs baseline: 1.5296x; 1.1431x over previous
import numpy as np
import jax
import jax.numpy as jnp
from jax.experimental import pallas as pl
from jax.experimental.pallas import tpu as pltpu

N = 10000
H = 256
B = 200
TD_MAX = 100.0
NPAD = 10240
BR = 8


def _scan_kernel(u_ref, v_ref, et_ref, td_ref, z0_ref, wa_ref, wb_ref,
                 WsT_ref, WrT_ref, WtT_ref, bias_ref,
                 srow_ref, a2_ref, z_ref, s_ref):
    j = pl.program_id(0)

    @pl.when(j == 0)
    def _init():
        z_ref[...] = z0_ref[...]
        s_ref[:, 0:N] = jax.lax.dot_general(
            wb_ref[...], z0_ref[...],
            dimension_numbers=(((1,), (1,)), ((), ())),
            preferred_element_type=jnp.float32)
        s_ref[:, N:NPAD] = jnp.zeros((2, NPAD - N), jnp.float32)

    def _event(i, r):
        ui = u_ref[i]
        vi = v_ref[i]
        et = et_ref[i]

        zu = z_ref[pl.ds(ui, 1), :]
        zv = z_ref[pl.ds(vi, 1), :]

        srow_ref[pl.ds(r, 1), :] = s_ref[pl.ds(et, 1), 0:N]
        wa_et = wa_ref[pl.ds(et, 1), :]
        a_u = jnp.sum(zu * wa_et, axis=1, keepdims=True)
        a_v = jnp.sum(zv * wa_et, axis=1, keepdims=True)
        a2_ref[pl.ds(r, 1), :] = jnp.concatenate([a_u, a_v], axis=1)

        tdn = td_ref[pl.ds(i, 1), :]
        td01 = jnp.concatenate([tdn[:, 0:4], tdn[:, 4:8]],
                               axis=0).astype(jnp.bfloat16)
        m_struct = jnp.concatenate([zv, zu], axis=0).astype(jnp.bfloat16)
        m_rec = jnp.concatenate([zu, zv], axis=0).astype(jnp.bfloat16)
        h2 = jax.nn.sigmoid(
            jnp.dot(m_struct, WsT_ref[...],
                    preferred_element_type=jnp.float32)
            + jnp.dot(m_rec, WrT_ref[...],
                      preferred_element_type=jnp.float32)
            + jnp.dot(td01, WtT_ref[...],
                      preferred_element_type=jnp.float32)
            + bias_ref[...])

        hu = h2[0:1, :]
        hv = h2[1:2, :]
        z_ref[pl.ds(ui, 1), :] = hu
        z_ref[pl.ds(vi, 1), :] = hv
        val_u = jnp.sum(wb_ref[...] * hu, axis=1, keepdims=True)
        val_v = jnp.sum(wb_ref[...] * hv, axis=1, keepdims=True)
        lane = jax.lax.broadcasted_iota(jnp.int32, (1, 128), 1)
        base_u = pl.multiple_of((ui // 128) * 128, 128)
        tile_u = s_ref[:, pl.ds(base_u, 128)]
        s_ref[:, pl.ds(base_u, 128)] = jnp.where(
            lane == ui % 128, val_u, tile_u)
        base_v = pl.multiple_of((vi // 128) * 128, 128)
        tile_v = s_ref[:, pl.ds(base_v, 128)]
        s_ref[:, pl.ds(base_v, 128)] = jnp.where(
            lane == vi % 128, val_v, tile_v)

    i0 = 4 * j
    r0 = i0 % 8
    _event(i0, r0)
    _event(i0 + 1, r0 + 1)
    _event(i0 + 2, r0 + 2)
    _event(i0 + 3, r0 + 3)


def _lam_kernel(wt_ref, al_ref, ps_ref, ob_ref, uv_ref, etc_ref, tf_ref,
                tb_ref, srow_ref, a2_ref, lam_ref):
    u_col = uv_ref[:, 0:1]
    v_col = uv_ref[:, 1:2]
    et_col = etc_ref[...]
    t_col = tf_ref[...]

    et0 = et_col == 0
    wt_col = jnp.where(et0, wt_ref[0], wt_ref[1])
    al_col = jnp.where(et0, al_ref[0], al_ref[1])
    ps_col = jnp.where(et0, ps_ref[0], ps_ref[1])
    ob_col = jnp.where(et0, ob_ref[0], ob_ref[1])
    c1 = wt_col * (1.0 / TD_MAX)
    c2 = jnp.log(al_col) - c1 * t_col
    inv_col = 1.0 / (ps_col + 1e-7)

    tb = tb_ref[:, 0, :]
    iota = jax.lax.broadcasted_iota(jnp.int32, (1, N), 1)
    tbu = jnp.sum(jnp.where(iota == u_col, tb, 0.0), axis=1, keepdims=True)
    tbv = jnp.sum(jnp.where(iota == v_col, tb, 0.0), axis=1, keepdims=True)
    dec_u = jnp.exp(c1 * jnp.maximum(tbu, tb) + c2)
    dec_v = jnp.exp(c1 * jnp.maximum(tbv, tb) + c2)
    base = srow_ref[...] + ob_col
    g_u = a2_ref[:, 0:1] + base + dec_u
    g_v = a2_ref[:, 1:2] + base + dec_v
    lam_ref[:, 0:N] = ps_col * jnp.log1p(
        jnp.exp(jnp.clip(g_u * inv_col, -75.0, 75.0)))
    lam_ref[:, N:2 * N] = ps_col * jnp.log1p(
        jnp.exp(jnp.clip(g_v * inv_col, -75.0, 75.0)))


@jax.jit
def kernel(u, v, time_diff, event_types, t_bar, t, z0, omega_w, omega_b,
           w_t, alpha, psi, W_struct_w, W_struct_b, W_rec_w, W_rec_b,
           W_t_w, W_t_b):
    u = u.astype(jnp.int32)
    v = v.astype(jnp.int32)
    et = event_types.astype(jnp.int32)
    td8 = time_diff.reshape(B, 8)
    wa = omega_w[:, 0, :H]
    wb = omega_w[:, 0, H:]
    ob = omega_b[:, 0]
    bias = (W_struct_b + W_rec_b + W_t_b).reshape(1, H)
    sd4 = jnp.array([50.0, 7.0, 15.0, 15.0], dtype=jnp.float32)
    WtT_n = W_t_w.T / sd4[:, None]
    WsT16 = W_struct_w.T.astype(jnp.bfloat16)
    WrT16 = W_rec_w.T.astype(jnp.bfloat16)
    WtT16 = WtT_n.astype(jnp.bfloat16)

    smem = pl.BlockSpec(memory_space=pltpu.SMEM)
    full = pl.BlockSpec(memory_space=pltpu.VMEM)

    srow2, a2, z_final = pl.pallas_call(
        _scan_kernel,
        grid=(B // 4,),
        in_specs=[
            smem,
            smem,
            smem,
            full,
            full,
            full,
            full,
            full,
            full,
            full,
            full,
        ],
        out_specs=[
            pl.BlockSpec((8, N), lambda j: (j // 2, 0)),
            pl.BlockSpec((8, 2), lambda j: (j // 2, 0)),
            pl.BlockSpec((N, H), lambda j: (0, 0)),
        ],
        out_shape=[
            jax.ShapeDtypeStruct((B, N), jnp.float32),
            jax.ShapeDtypeStruct((B, 2), jnp.float32),
            jax.ShapeDtypeStruct((N, H), jnp.float32),
        ],
        scratch_shapes=[pltpu.VMEM((2, NPAD), jnp.float32)],
        compiler_params=pltpu.CompilerParams(
            dimension_semantics=("arbitrary",)),
    )(u, v, et, td8, z0, wa, wb, WsT16, WrT16, WtT16, bias)

    uv = jnp.stack([u, v], axis=1)
    lam = pl.pallas_call(
        _lam_kernel,
        grid=(B // BR,),
        in_specs=[
            smem,
            smem,
            smem,
            smem,
            pl.BlockSpec((BR, 2), lambda b: (b, 0)),
            pl.BlockSpec((BR, 1), lambda b: (b, 0)),
            pl.BlockSpec((BR, 1), lambda b: (b, 0)),
            pl.BlockSpec((BR, 1, N), lambda b: (b, 0, 0)),
            pl.BlockSpec((BR, N), lambda b: (b, 0)),
            pl.BlockSpec((BR, 2), lambda b: (b, 0)),
        ],
        out_specs=pl.BlockSpec((BR, 2 * N), lambda b: (b, 0)),
        out_shape=jax.ShapeDtypeStruct((B, 2 * N), jnp.float32),
        compiler_params=pltpu.CompilerParams(
            dimension_semantics=("parallel",)),
    )(w_t, alpha, psi, ob, uv, et.reshape(B, 1), t.reshape(B, 1),
      t_bar.reshape(B, 1, N), srow2, a2)

    return lam, z_final


# speedup vs baseline: 1.5997x; 1.0458x over previous
import numpy as np
import jax
import jax.numpy as jnp
from jax.experimental import pallas as pl
from jax.experimental.pallas import tpu as pltpu

N = 10000
H = 256
B = 200
TD_MAX = 100.0
NPAD = 10240
BR = 8


def _scan_kernel(u_ref, v_ref, et_ref, td_ref, z0_ref, wa_ref, wb_ref,
                 WsT_ref, WrT_ref, WtT_ref, bias_ref,
                 srow_ref, a2_ref, z_ref, s_ref):
    j = pl.program_id(0)

    @pl.when(j == 0)
    def _init():
        z_ref[...] = z0_ref[...]
        s_ref[:, 0:N] = jax.lax.dot_general(
            wb_ref[...], z0_ref[...],
            dimension_numbers=(((1,), (1,)), ((), ())),
            preferred_element_type=jnp.float32)
        s_ref[:, N:NPAD] = jnp.zeros((2, NPAD - N), jnp.float32)

    def _event(i, r):
        ui = u_ref[i]
        vi = v_ref[i]
        et = et_ref[i]

        zu = z_ref[pl.ds(ui, 1), :]
        zv = z_ref[pl.ds(vi, 1), :]

        srow_ref[pl.ds(r, 1), :] = s_ref[pl.ds(et, 1), 0:N]
        wa_et = wa_ref[pl.ds(et, 1), :]
        a_u = jnp.sum(zu * wa_et, axis=1, keepdims=True)
        a_v = jnp.sum(zv * wa_et, axis=1, keepdims=True)
        a2_ref[pl.ds(r, 1), :] = jnp.concatenate([a_u, a_v], axis=1)

        tdn = td_ref[pl.ds(i, 1), :]
        td01 = jnp.concatenate([tdn[:, 0:4], tdn[:, 4:8]],
                               axis=0).astype(jnp.bfloat16)
        m_struct = jnp.concatenate([zv, zu], axis=0).astype(jnp.bfloat16)
        m_rec = jnp.concatenate([zu, zv], axis=0).astype(jnp.bfloat16)
        h2 = jax.nn.sigmoid(
            jnp.dot(m_struct, WsT_ref[...],
                    preferred_element_type=jnp.float32)
            + jnp.dot(m_rec, WrT_ref[...],
                      preferred_element_type=jnp.float32)
            + jnp.dot(td01, WtT_ref[...],
                      preferred_element_type=jnp.float32)
            + bias_ref[...])

        hu = h2[0:1, :]
        hv = h2[1:2, :]
        z_ref[pl.ds(ui, 1), :] = hu
        z_ref[pl.ds(vi, 1), :] = hv
        val_u = jnp.sum(wb_ref[...] * hu, axis=1, keepdims=True)
        val_v = jnp.sum(wb_ref[...] * hv, axis=1, keepdims=True)
        lane = jax.lax.broadcasted_iota(jnp.int32, (1, 128), 1)
        base_u = pl.multiple_of((ui // 128) * 128, 128)
        tile_u = s_ref[:, pl.ds(base_u, 128)]
        s_ref[:, pl.ds(base_u, 128)] = jnp.where(
            lane == ui % 128, val_u, tile_u)
        base_v = pl.multiple_of((vi // 128) * 128, 128)
        tile_v = s_ref[:, pl.ds(base_v, 128)]
        s_ref[:, pl.ds(base_v, 128)] = jnp.where(
            lane == vi % 128, val_v, tile_v)

    i0 = 8 * j
    for k in range(8):
        _event(i0 + k, k)


def _lam_kernel(wt_ref, al_ref, ps_ref, ob_ref, uv_ref, etc_ref, tf_ref,
                tb_ref, srow_ref, a2_ref, lam_ref):
    u_col = uv_ref[:, 0:1]
    v_col = uv_ref[:, 1:2]
    et_col = etc_ref[...]
    t_col = tf_ref[...]

    et0 = et_col == 0
    wt_col = jnp.where(et0, wt_ref[0], wt_ref[1])
    al_col = jnp.where(et0, al_ref[0], al_ref[1])
    ps_col = jnp.where(et0, ps_ref[0], ps_ref[1])
    ob_col = jnp.where(et0, ob_ref[0], ob_ref[1])
    c1 = wt_col * (1.0 / TD_MAX)
    c2 = jnp.log(al_col) - c1 * t_col
    inv_col = 1.0 / (ps_col + 1e-7)

    tb = tb_ref[:, 0, :]
    iota = jax.lax.broadcasted_iota(jnp.int32, (1, N), 1)
    tbu = jnp.sum(jnp.where(iota == u_col, tb, 0.0), axis=1, keepdims=True)
    tbv = jnp.sum(jnp.where(iota == v_col, tb, 0.0), axis=1, keepdims=True)
    dec_u = jnp.exp(c1 * jnp.maximum(tbu, tb) + c2)
    dec_v = jnp.exp(c1 * jnp.maximum(tbv, tb) + c2)
    base = srow_ref[...] + ob_col
    g_u = a2_ref[:, 0:1] + base + dec_u
    g_v = a2_ref[:, 1:2] + base + dec_v
    lam_ref[:, 0:N] = ps_col * jnp.log1p(
        jnp.exp(jnp.clip(g_u * inv_col, -75.0, 75.0)))
    lam_ref[:, N:2 * N] = ps_col * jnp.log1p(
        jnp.exp(jnp.clip(g_v * inv_col, -75.0, 75.0)))


@jax.jit
def kernel(u, v, time_diff, event_types, t_bar, t, z0, omega_w, omega_b,
           w_t, alpha, psi, W_struct_w, W_struct_b, W_rec_w, W_rec_b,
           W_t_w, W_t_b):
    u = u.astype(jnp.int32)
    v = v.astype(jnp.int32)
    et = event_types.astype(jnp.int32)
    td8 = time_diff.reshape(B, 8)
    wa = omega_w[:, 0, :H]
    wb = omega_w[:, 0, H:]
    ob = omega_b[:, 0]
    bias = (W_struct_b + W_rec_b + W_t_b).reshape(1, H)
    sd4 = jnp.array([50.0, 7.0, 15.0, 15.0], dtype=jnp.float32)
    WtT_n = W_t_w.T / sd4[:, None]
    WsT16 = W_struct_w.T.astype(jnp.bfloat16)
    WrT16 = W_rec_w.T.astype(jnp.bfloat16)
    WtT16 = WtT_n.astype(jnp.bfloat16)

    smem = pl.BlockSpec(memory_space=pltpu.SMEM)
    full = pl.BlockSpec(memory_space=pltpu.VMEM)

    srow2, a2, z_final = pl.pallas_call(
        _scan_kernel,
        grid=(B // 8,),
        in_specs=[
            smem,
            smem,
            smem,
            full,
            full,
            full,
            full,
            full,
            full,
            full,
            full,
        ],
        out_specs=[
            pl.BlockSpec((8, N), lambda j: (j, 0)),
            pl.BlockSpec((8, 2), lambda j: (j, 0)),
            pl.BlockSpec((N, H), lambda j: (0, 0)),
        ],
        out_shape=[
            jax.ShapeDtypeStruct((B, N), jnp.float32),
            jax.ShapeDtypeStruct((B, 2), jnp.float32),
            jax.ShapeDtypeStruct((N, H), jnp.float32),
        ],
        scratch_shapes=[pltpu.VMEM((2, NPAD), jnp.float32)],
        compiler_params=pltpu.CompilerParams(
            dimension_semantics=("arbitrary",)),
    )(u, v, et, td8, z0, wa, wb, WsT16, WrT16, WtT16, bias)

    uv = jnp.stack([u, v], axis=1)
    lam = pl.pallas_call(
        _lam_kernel,
        grid=(B // BR,),
        in_specs=[
            smem,
            smem,
            smem,
            smem,
            pl.BlockSpec((BR, 2), lambda b: (b, 0)),
            pl.BlockSpec((BR, 1), lambda b: (b, 0)),
            pl.BlockSpec((BR, 1), lambda b: (b, 0)),
            pl.BlockSpec((BR, 1, N), lambda b: (b, 0, 0)),
            pl.BlockSpec((BR, N), lambda b: (b, 0)),
            pl.BlockSpec((BR, 2), lambda b: (b, 0)),
        ],
        out_specs=pl.BlockSpec((BR, 2 * N), lambda b: (b, 0)),
        out_shape=jax.ShapeDtypeStruct((B, 2 * N), jnp.float32),
        compiler_params=pltpu.CompilerParams(
            dimension_semantics=("parallel",)),
    )(w_t, alpha, psi, ob, uv, et.reshape(B, 1), t.reshape(B, 1),
      t_bar.reshape(B, 1, N), srow2, a2)

    return lam, z_final
